# baseline (speedup 1.0000x reference)
import functools

import jax
import jax.numpy as jnp
from jax.experimental import pallas as pl

RATIO = 16


@functools.lru_cache(maxsize=None)
def _make_tc_call(B: int, Sl: int, d: int, grid: int):
    Sg = Sl // RATIO
    n = B * Sg
    dblk = d // grid

    def body(mask_ref, emb_ref, out_ref, flag_ref):
        flags = jnp.all(mask_ref[...], axis=2)
        flag_ref[...] = flags
        keep = 1.0 - flags.astype(jnp.float32)
        out_ref[...] = keep[:, :, None] * emb_ref[1, :][None, None, :]

    return pl.pallas_call(
        body,
        grid=(grid,),
        in_specs=[
            pl.BlockSpec((B, Sg, RATIO), lambda i: (0, 0, 0)),
            pl.BlockSpec((2, dblk), lambda i: (0, i)),
        ],
        out_specs=[
            pl.BlockSpec((B, Sg, dblk), lambda i: (0, 0, i)),
            pl.BlockSpec((B, Sg), lambda i: (0, 0)),
        ],
        out_shape=[
            jax.ShapeDtypeStruct((B, Sg, d), jnp.float32),
            jax.ShapeDtypeStruct((B, Sg), jnp.bool_),
        ],
    )


def kernel(token_ids, padding_mask, embeds):
    B, Sl = padding_mask.shape
    d = embeds.shape[1]
    Sg = Sl // RATIO
    out, flags = _make_tc_call(B, Sl, d, 2)(
        padding_mask.reshape(B, Sg, RATIO), embeds)
    return out, flags


# speedup vs baseline: 1.0221x; 1.0221x over previous
import functools

import jax
import jax.numpy as jnp
from jax.experimental import pallas as pl

RATIO = 16


def _body(mask_ref, emb_ref, out_ref, flag_ref):
    flags = jnp.all(mask_ref[...], axis=2)
    flag_ref[...] = flags
    keep = 1.0 - flags.astype(jnp.float32)
    out_ref[...] = keep[:, :, None] * emb_ref[1, :][None, None, :]


@functools.lru_cache(maxsize=None)
def _make_tc_call(B: int, Sl: int, d: int, grid: int):
    Sg = Sl // RATIO
    sblk = Sg // grid
    return pl.pallas_call(
        _body,
        grid=(grid,),
        in_specs=[
            pl.BlockSpec((B, sblk, RATIO), lambda i: (0, i, 0)),
            pl.BlockSpec((2, d), lambda i: (0, 0)),
        ],
        out_specs=[
            pl.BlockSpec((B, sblk, d), lambda i: (0, i, 0)),
            pl.BlockSpec((B, sblk), lambda i: (0, i)),
        ],
        out_shape=[
            jax.ShapeDtypeStruct((B, Sg, d), jnp.float32),
            jax.ShapeDtypeStruct((B, Sg), jnp.bool_),
        ],
    )


def kernel(token_ids, padding_mask, embeds):
    B, Sl = padding_mask.shape
    d = embeds.shape[1]
    Sg = Sl // RATIO
    return _make_tc_call(B, Sl, d, 2)(
        padding_mask.reshape(B, Sg, RATIO), embeds)
